# Initial kernel scaffold; baseline (speedup 1.0000x reference)
import functools

import jax
import jax.numpy as jnp
from jax import lax
from jax.experimental import pallas as pl
from jax.experimental.pallas import tpu as pltpu
from jax.experimental.pallas import tpu_sc as plsc

N_NODES = 10000
N_EDGES = 320000
D = 128

NC = 2
NS = 16
NW = NC * NS

E_PER_TILE = N_EDGES // NW
CHUNK = 80
NCHUNK = E_PER_TILE // CHUNK
ROWS_PER_TILE = N_NODES // NS
DEG_W = 16

_mesh = plsc.VectorSubcoreMesh(core_axis_name="c", subcore_axis_name="s")


@functools.partial(
    pl.kernel,
    out_type=(
        jax.ShapeDtypeStruct((NC, N_NODES, D), jnp.float32),
        jax.ShapeDtypeStruct((NC, N_NODES, DEG_W), jnp.float32),
    ),
    mesh=_mesh,
    scratch_types=[
        pltpu.VMEM_SHARED((N_NODES, D), jnp.float32),
        pltpu.VMEM_SHARED((N_NODES, DEG_W), jnp.float32),
        pltpu.VMEM((NCHUNK, CHUNK), jnp.int32),
        pltpu.VMEM((NCHUNK, CHUNK), jnp.int32),
        pltpu.VMEM((CHUNK, D), jnp.float32),
        pltpu.VMEM((CHUNK, DEG_W), jnp.float32),
        pltpu.SemaphoreType.DMA,
    ],
)
def _sc_segment_sum(data_h, src_h, dst_h, z128_h, zdeg_h, ones_h,
                    acc_out_h, deg_out_h,
                    acc_s, deg_s, sidx_v, didx_v, rows_v, ones_v, sem):
    c = lax.axis_index("c")
    s = lax.axis_index("s")
    wid = c * NS + s

    pltpu.sync_copy(src_h.at[pl.ds(wid * NCHUNK, NCHUNK)], sidx_v)
    pltpu.sync_copy(dst_h.at[pl.ds(wid * NCHUNK, NCHUNK)], didx_v)
    pltpu.sync_copy(ones_h, ones_v)

    r0 = s * ROWS_PER_TILE
    pltpu.sync_copy(z128_h, acc_s.at[pl.ds(r0, ROWS_PER_TILE)])
    pltpu.sync_copy(zdeg_h, deg_s.at[pl.ds(r0, ROWS_PER_TILE)])
    plsc.subcore_barrier()

    def chunk_body(j, carry):
        pltpu.async_copy(data_h.at[sidx_v.at[j]], rows_v, sem).wait()
        pltpu.sync_copy(rows_v, acc_s.at[didx_v.at[j]], add=True)
        pltpu.sync_copy(ones_v, deg_s.at[didx_v.at[j]], add=True)
        return carry

    lax.fori_loop(0, NCHUNK, chunk_body, 0)
    plsc.subcore_barrier()

    pltpu.sync_copy(acc_s.at[pl.ds(r0, ROWS_PER_TILE)],
                    acc_out_h.at[c, pl.ds(r0, ROWS_PER_TILE)])
    pltpu.sync_copy(deg_s.at[pl.ds(r0, ROWS_PER_TILE)],
                    deg_out_h.at[c, pl.ds(r0, ROWS_PER_TILE)])


BLK = 1000


def _tc_combine_body(d_ref, acc_ref, deg_ref, wt_ref, b_ref, o_ref):
    deg = deg_ref[0, :, 0:1] + deg_ref[1, :, 0:1]
    z = (d_ref[...] + acc_ref[0] + acc_ref[1]) / (deg + 1.0)
    y = jnp.dot(z, wt_ref[...], preferred_element_type=jnp.float32) + b_ref[...]
    o_ref[...] = jnp.where(deg > 0.0, jnp.maximum(y, 0.0), 0.0)


_tc_combine = pl.pallas_call(
    _tc_combine_body,
    grid=(N_NODES // BLK,),
    in_specs=[
        pl.BlockSpec((BLK, D), lambda i: (i, 0)),
        pl.BlockSpec((NC, BLK, D), lambda i: (0, i, 0)),
        pl.BlockSpec((NC, BLK, DEG_W), lambda i: (0, i, 0)),
        pl.BlockSpec((D, D), lambda i: (0, 0)),
        pl.BlockSpec((1, D), lambda i: (0, 0)),
    ],
    out_specs=pl.BlockSpec((BLK, D), lambda i: (i, 0)),
    out_shape=jax.ShapeDtypeStruct((N_NODES, D), jnp.float32),
)


@jax.jit
def kernel(data, structure, W, b):
    src = structure[0].reshape(NW * NCHUNK, CHUNK)
    dst = structure[1].reshape(NW * NCHUNK, CHUNK)
    z128 = jnp.zeros((ROWS_PER_TILE, D), jnp.float32)
    zdeg = jnp.zeros((ROWS_PER_TILE, DEG_W), jnp.float32)
    ones = jnp.ones((CHUNK, DEG_W), jnp.float32)
    acc, deg = _sc_segment_sum(data, src, dst, z128, zdeg, ones)
    return _tc_combine(data, acc, deg, W.T, b.reshape(1, D))


# speedup vs baseline: 6.9956x; 6.9956x over previous
import functools

import jax
import jax.numpy as jnp
from jax import lax
from jax.experimental import pallas as pl
from jax.experimental.pallas import tpu as pltpu
from jax.experimental.pallas import tpu_sc as plsc

N_NODES = 10000
N_EDGES = 320000
D = 128
DE = 144

NC = 2
NS = 16
NW = NC * NS

E_PER_TILE = N_EDGES // NW
CHUNK = 50
SLAB = 40
NSLAB = E_PER_TILE // (SLAB * CHUNK)
N_PAD = 10240
ROWS_PER_TILE = N_PAD // NS

_mesh = plsc.VectorSubcoreMesh(core_axis_name="c", subcore_axis_name="s")


@functools.partial(
    pl.kernel,
    out_type=jax.ShapeDtypeStruct((NC, N_PAD, DE), jnp.float32),
    mesh=_mesh,
    compiler_params=pltpu.CompilerParams(use_tc_tiling_on_sc=False),
    scratch_types=[
        pltpu.VMEM_SHARED((N_PAD, DE), jnp.float32),
        pltpu.VMEM((SLAB, CHUNK), jnp.int32),
        pltpu.VMEM((SLAB, CHUNK), jnp.int32),
        pltpu.VMEM((CHUNK, DE), jnp.float32),
        pltpu.SemaphoreType.DMA,
    ],
)
def _sc_segment_sum(data_h, src_h, dst_h, zero_h,
                    acc_out_h,
                    acc_s, sidx_v, didx_v, rows_v, sem):
    c = lax.axis_index("c")
    s = lax.axis_index("s")
    wid = c * NS + s

    r0 = s * ROWS_PER_TILE
    pltpu.sync_copy(zero_h, acc_s.at[pl.ds(r0, ROWS_PER_TILE)])
    plsc.subcore_barrier()

    def slab_body(t, carry):
        pltpu.sync_copy(src_h.at[wid, t], sidx_v)
        pltpu.sync_copy(dst_h.at[wid, t], didx_v)

        def chunk_body(r, carry2):
            pltpu.async_copy(data_h.at[sidx_v.at[r]], rows_v, sem).wait()
            pltpu.sync_copy(rows_v, acc_s.at[didx_v.at[r]], add=True)
            return carry2

        lax.fori_loop(0, SLAB, chunk_body, 0)
        return carry

    lax.fori_loop(0, NSLAB, slab_body, 0)
    plsc.subcore_barrier()

    pltpu.sync_copy(acc_s.at[pl.ds(r0, ROWS_PER_TILE)],
                    acc_out_h.at[c, pl.ds(r0, ROWS_PER_TILE)])


BLK = 1000


def _tc_combine_body(d_ref, acc_ref, deg_ref, wt_ref, b_ref, o_ref):
    deg = deg_ref[...]
    z = (d_ref[...] + acc_ref[0] + acc_ref[1]) / (deg + 1.0)
    y = jnp.dot(z, wt_ref[...], preferred_element_type=jnp.float32) + b_ref[...]
    o_ref[...] = jnp.where(deg > 0.0, jnp.maximum(y, 0.0), 0.0)


_tc_combine = pl.pallas_call(
    _tc_combine_body,
    grid=(N_NODES // BLK,),
    in_specs=[
        pl.BlockSpec((BLK, D), lambda i: (i, 0)),
        pl.BlockSpec((NC, BLK, D), lambda i: (0, i, 0)),
        pl.BlockSpec((BLK, 1), lambda i: (i, 0)),
        pl.BlockSpec((D, D), lambda i: (0, 0)),
        pl.BlockSpec((1, D), lambda i: (0, 0)),
    ],
    out_specs=pl.BlockSpec((BLK, D), lambda i: (i, 0)),
    out_shape=jax.ShapeDtypeStruct((N_NODES, D), jnp.float32),
)


@jax.jit
def kernel(data, structure, W, b):
    src = structure[0].reshape(NW, NSLAB, SLAB, CHUNK)
    dst = structure[1].reshape(NW, NSLAB, SLAB, CHUNK)
    data_ext = jnp.concatenate(
        [data, jnp.ones((N_NODES, DE - D), jnp.float32)], axis=1)
    zero = jnp.zeros((ROWS_PER_TILE, DE), jnp.float32)
    acc = _sc_segment_sum(data_ext, src, dst, zero)
    accm = acc[:, :, :D]
    deg_col = (acc[0, :, D] + acc[1, :, D]).reshape(N_PAD, 1)
    return _tc_combine(data, accm, deg_col, W.T, b.reshape(1, D))


# speedup vs baseline: 9.9091x; 1.4165x over previous
import functools

import jax
import jax.numpy as jnp
from jax import lax
from jax.experimental import pallas as pl
from jax.experimental.pallas import tpu as pltpu
from jax.experimental.pallas import tpu_sc as plsc

N_NODES = 10000
N_EDGES = 320000
D = 128
DE = 144

NC = 2
NS = 16
NW = NC * NS

E_PER_TILE = N_EDGES // NW
CHUNK = 125
SLAB = 20
NSLAB = E_PER_TILE // (SLAB * CHUNK)
N_PAD = 10240
ROWS_PER_TILE = N_PAD // NS

_mesh = plsc.VectorSubcoreMesh(core_axis_name="c", subcore_axis_name="s")


@functools.partial(
    pl.kernel,
    out_type=jax.ShapeDtypeStruct((NC, N_PAD, DE), jnp.float32),
    mesh=_mesh,
    compiler_params=pltpu.CompilerParams(use_tc_tiling_on_sc=False),
    scratch_types=[
        pltpu.VMEM_SHARED((N_PAD, DE), jnp.float32),
        pltpu.VMEM((SLAB, CHUNK), jnp.int32),
        pltpu.VMEM((SLAB, CHUNK), jnp.int32),
        pltpu.VMEM((CHUNK, DE), jnp.float32),
        pltpu.SemaphoreType.DMA,
    ],
)
def _sc_segment_sum(data_h, src_h, dst_h, zero_h,
                    acc_out_h,
                    acc_s, sidx_v, didx_v, rows_v, sem):
    c = lax.axis_index("c")
    s = lax.axis_index("s")
    wid = c * NS + s

    r0 = s * ROWS_PER_TILE
    pltpu.sync_copy(zero_h, acc_s.at[pl.ds(r0, ROWS_PER_TILE)])
    plsc.subcore_barrier()

    def slab_body(t, carry):
        pltpu.sync_copy(src_h.at[wid, t], sidx_v)
        pltpu.sync_copy(dst_h.at[wid, t], didx_v)

        def chunk_body(r, carry2):
            pltpu.async_copy(data_h.at[sidx_v.at[r]], rows_v, sem).wait()
            pltpu.sync_copy(rows_v, acc_s.at[didx_v.at[r]], add=True)
            return carry2

        lax.fori_loop(0, SLAB, chunk_body, 0)
        return carry

    lax.fori_loop(0, NSLAB, slab_body, 0)
    plsc.subcore_barrier()

    pltpu.sync_copy(acc_s.at[pl.ds(r0, ROWS_PER_TILE)],
                    acc_out_h.at[c, pl.ds(r0, ROWS_PER_TILE)])


BLK = 1000


def _tc_combine_body(d_ref, acc_ref, deg_ref, wt_ref, b_ref, o_ref):
    deg = deg_ref[...]
    z = (d_ref[...] + acc_ref[0] + acc_ref[1]) / (deg + 1.0)
    y = jnp.dot(z, wt_ref[...], preferred_element_type=jnp.float32) + b_ref[...]
    o_ref[...] = jnp.where(deg > 0.0, jnp.maximum(y, 0.0), 0.0)


_tc_combine = pl.pallas_call(
    _tc_combine_body,
    grid=(N_NODES // BLK,),
    in_specs=[
        pl.BlockSpec((BLK, D), lambda i: (i, 0)),
        pl.BlockSpec((NC, BLK, D), lambda i: (0, i, 0)),
        pl.BlockSpec((BLK, 1), lambda i: (i, 0)),
        pl.BlockSpec((D, D), lambda i: (0, 0)),
        pl.BlockSpec((1, D), lambda i: (0, 0)),
    ],
    out_specs=pl.BlockSpec((BLK, D), lambda i: (i, 0)),
    out_shape=jax.ShapeDtypeStruct((N_NODES, D), jnp.float32),
)


@jax.jit
def kernel(data, structure, W, b):
    src = structure[0].reshape(NW, NSLAB, SLAB, CHUNK)
    dst = structure[1].reshape(NW, NSLAB, SLAB, CHUNK)
    data_ext = jnp.concatenate(
        [data, jnp.ones((N_NODES, DE - D), jnp.float32)], axis=1)
    zero = jnp.zeros((ROWS_PER_TILE, DE), jnp.float32)
    acc = _sc_segment_sum(data_ext, src, dst, zero)
    accm = acc[:, :, :D]
    deg_col = (acc[0, :, D] + acc[1, :, D]).reshape(N_PAD, 1)
    return _tc_combine(data, accm, deg_col, W.T, b.reshape(1, D))
